# Initial kernel scaffold; baseline (speedup 1.0000x reference)
import jax
import jax.numpy as jnp
from jax.experimental import pallas as pl


def kernel(node, connect, bond, bond_neighbour, W_node_w, W_node_b, W_node_final_w, W_node_final_b, W_bond_w, W_bond_b, W_bond_final_w, W_bond_final_b, W_z_w, W_z_b, W_r_w, W_r_b, U_w, W_w, W_b, W_n_w, W_n_b, U_n_w):
    raise NotImplementedError("write your pallas kernel here")


# speedup vs baseline: 1.0659x; 1.0659x over previous
import functools

import jax
import jax.numpy as jnp
from jax import lax
from jax.experimental import pallas as pl
from jax.experimental.pallas import tpu as pltpu

_LAYER = 3
_D = 128


def _hswish(x):
    return x * jnp.clip(x + 3.0, 0.0, 6.0) / 6.0


def _mm_body(act, nx, *refs):
    in_refs = refs[:nx]
    w_refs = refs[nx:2 * nx]
    b_ref = refs[2 * nx]
    o_ref = refs[2 * nx + 1]
    acc = b_ref[...].astype(jnp.float32)
    for x_ref, w_ref in zip(in_refs, w_refs):
        acc = acc + jnp.dot(x_ref[...], w_ref[...],
                            preferred_element_type=jnp.float32)
    if act == "hswish":
        acc = _hswish(acc)
    elif act == "sigmoid":
        acc = jax.nn.sigmoid(acc)
    o_ref[...] = acc


def _mm_fused(xs, ws, bias, act, block_rows=2000):
    rows = xs[0].shape[0]
    grid = (rows // block_rows,)
    nx = len(xs)
    in_specs = (
        [pl.BlockSpec((block_rows, x.shape[1]), lambda i: (i, 0)) for x in xs]
        + [pl.BlockSpec(w.shape, lambda i: (0, 0)) for w in ws]
        + [pl.BlockSpec((1, _D), lambda i: (0, 0))]
    )
    return pl.pallas_call(
        functools.partial(_mm_body, act, nx),
        grid=grid,
        in_specs=in_specs,
        out_specs=pl.BlockSpec((block_rows, _D), lambda i: (i, 0)),
        out_shape=jax.ShapeDtypeStruct((rows, _D), jnp.float32),
    )(*xs, *ws, bias.reshape(1, _D))


def _rki_body(pre_ref, mk_ref, w_ref, o_ref):
    mk = mk_ref[...]
    r = jax.nn.sigmoid(pre_ref[...] + jnp.dot(mk, w_ref[...],
                                              preferred_element_type=jnp.float32))
    o_ref[...] = r * mk


def _rki_fused(pre, mk, w, block_rows=2000):
    rows = pre.shape[0]
    return pl.pallas_call(
        _rki_body,
        grid=(rows // block_rows,),
        in_specs=[
            pl.BlockSpec((block_rows, _D), lambda i: (i, 0)),
            pl.BlockSpec((block_rows, _D), lambda i: (i, 0)),
            pl.BlockSpec((_D, _D), lambda i: (0, 0)),
        ],
        out_specs=pl.BlockSpec((block_rows, _D), lambda i: (i, 0)),
        out_shape=jax.ShapeDtypeStruct((rows, _D), jnp.float32),
    )(pre, mk, w)


def _bond_upd_body(pre_z_ref, pre_m_ref, s_ref, r_ref, wz_ref, uw_ref, o_ref):
    s = s_ref[...]
    z = jax.nn.sigmoid(pre_z_ref[...] + jnp.dot(s, wz_ref[...],
                                                preferred_element_type=jnp.float32))
    m = jnp.tanh(pre_m_ref[...] + jnp.dot(r_ref[...], uw_ref[...],
                                          preferred_element_type=jnp.float32))
    o_ref[...] = (1.0 - z) * s + z * m


def _bond_upd(pre_z, pre_m, s, r, wz, uw, block_rows=2000):
    rows = pre_z.shape[0]
    bs = lambda: pl.BlockSpec((block_rows, _D), lambda i: (i, 0))
    return pl.pallas_call(
        _bond_upd_body,
        grid=(rows // block_rows,),
        in_specs=[bs(), bs(), bs(), bs(),
                  pl.BlockSpec((_D, _D), lambda i: (0, 0)),
                  pl.BlockSpec((_D, _D), lambda i: (0, 0))],
        out_specs=bs(),
        out_shape=jax.ShapeDtypeStruct((rows, _D), jnp.float32),
    )(pre_z, pre_m, s, r, wz, uw)


def _node_upd_body(pre_ref, mn_ref, ag_ref, u1_ref, u2_ref, o_ref):
    acc = pre_ref[...]
    acc = acc + jnp.dot(mn_ref[...], u1_ref[...], preferred_element_type=jnp.float32)
    acc = acc + jnp.dot(ag_ref[...], u2_ref[...], preferred_element_type=jnp.float32)
    o_ref[...] = _hswish(acc)


def _node_upd(pre_n, mn, aggr, u1, u2, block_rows=2000):
    rows = pre_n.shape[0]
    bs = lambda: pl.BlockSpec((block_rows, _D), lambda i: (i, 0))
    return pl.pallas_call(
        _node_upd_body,
        grid=(rows // block_rows,),
        in_specs=[bs(), bs(), bs(),
                  pl.BlockSpec((_D, _D), lambda i: (0, 0)),
                  pl.BlockSpec((_D, _D), lambda i: (0, 0))],
        out_specs=bs(),
        out_shape=jax.ShapeDtypeStruct((rows, _D), jnp.float32),
    )(pre_n, mn, aggr, u1, u2)


def _gather_rows(table, idx):
    return jnp.take(table, idx, axis=0)


def _segment_sum(vals, idx, num_segments):
    return jax.ops.segment_sum(vals, idx, num_segments=num_segments)


def kernel(node, connect, bond, bond_neighbour, W_node_w, W_node_b,
           W_node_final_w, W_node_final_b, W_bond_w, W_bond_b,
           W_bond_final_w, W_bond_final_b, W_z_w, W_z_b, W_r_w, W_r_b,
           U_w, W_w, W_b, W_n_w, W_n_b, U_n_w):
    i_idx = connect[0]
    j_idx = connect[1]
    ij_idx = bond_neighbour[0]
    ki_idx = bond_neighbour[1]
    N = node.shape[0]
    E = bond.shape[0]
    FN = node.shape[1]
    FB = bond.shape[1]

    nodei = _gather_rows(node, i_idx)

    mess_bond = _mm_fused([nodei, bond], [W_bond_w[:FN], W_bond_w[FN:]],
                          W_bond_b, "hswish")
    mess_node = _mm_fused([node], [W_node_w], W_node_b, "hswish",
                          block_rows=2000)
    pre_z = _mm_fused([nodei, bond], [W_z_w[:FN], W_z_w[FN:FN + FB]],
                      W_z_b, "none")
    pre_m = _mm_fused([nodei, bond], [W_w[:FN], W_w[FN:]], W_b, "none")
    pre_n = _mm_fused([node], [W_n_w], W_n_b, "none", block_rows=2000)

    gi = _gather_rows(nodei, ij_idx)
    gb = _gather_rows(bond, ij_idx)
    pre_r = _mm_fused([gi, gb], [W_r_w[:FN], W_r_w[FN:FN + FB]],
                      W_r_b, "none")

    wz2 = W_z_w[FN + FB:]
    wr2 = W_r_w[FN + FB:]
    un1 = U_n_w[:_D]
    un2 = U_n_w[_D:]

    for _ in range(_LAYER):
        mess_ki = _gather_rows(mess_bond, ki_idx)
        s_ij = _segment_sum(mess_ki, ij_idx, E)
        rmk = _rki_fused(pre_r, mess_ki, wr2)
        r_ij = _segment_sum(rmk, ij_idx, E)
        mess_bond = _bond_upd(pre_z, pre_m, s_ij, r_ij, wz2, U_w)
        aggr_node = _segment_sum(mess_bond, j_idx, N)
        mess_node = _node_upd(pre_n, mess_node, aggr_node, un1, un2)

    out_bond = _mm_fused([nodei, bond, mess_bond],
                         [W_bond_final_w[:FN], W_bond_final_w[FN:FN + FB],
                          W_bond_final_w[FN + FB:]],
                         W_bond_final_b, "hswish")
    out_node = _mm_fused([node, mess_node],
                         [W_node_final_w[:FN], W_node_final_w[FN:]],
                         W_node_final_b, "hswish", block_rows=2000)
    return (out_node, out_bond)


# speedup vs baseline: 1.1857x; 1.1124x over previous
import functools

import jax
import jax.numpy as jnp
from jax import lax
from jax.experimental import pallas as pl
from jax.experimental.pallas import tpu as pltpu
from jax.experimental.pallas import tpu_sc as plsc

_LAYER = 3
_D = 128
_NC, _NS = 2, 16
_NW = _NC * _NS


def _hswish(x):
    return x * jnp.clip(x + 3.0, 0.0, 6.0) / 6.0


def _mm_body(act, nx, *refs):
    in_refs = refs[:nx]
    w_refs = refs[nx:2 * nx]
    b_ref = refs[2 * nx]
    o_ref = refs[2 * nx + 1]
    acc = b_ref[...].astype(jnp.float32)
    for x_ref, w_ref in zip(in_refs, w_refs):
        acc = acc + jnp.dot(x_ref[...], w_ref[...],
                            preferred_element_type=jnp.float32)
    if act == "hswish":
        acc = _hswish(acc)
    elif act == "sigmoid":
        acc = jax.nn.sigmoid(acc)
    o_ref[...] = acc


def _mm_fused(xs, ws, bias, act, block_rows=2000):
    rows = xs[0].shape[0]
    grid = (rows // block_rows,)
    nx = len(xs)
    in_specs = (
        [pl.BlockSpec((block_rows, x.shape[1]), lambda i: (i, 0)) for x in xs]
        + [pl.BlockSpec(w.shape, lambda i: (0, 0)) for w in ws]
        + [pl.BlockSpec((1, _D), lambda i: (0, 0))]
    )
    return pl.pallas_call(
        functools.partial(_mm_body, act, nx),
        grid=grid,
        in_specs=in_specs,
        out_specs=pl.BlockSpec((block_rows, _D), lambda i: (i, 0)),
        out_shape=jax.ShapeDtypeStruct((rows, _D), jnp.float32),
    )(*xs, *ws, bias.reshape(1, _D))


def _rki_body(pre_ref, mk_ref, w_ref, o_ref):
    mk = mk_ref[...]
    r = jax.nn.sigmoid(pre_ref[...] + jnp.dot(mk, w_ref[...],
                                              preferred_element_type=jnp.float32))
    o_ref[...] = r * mk


def _rki_fused(pre, mk, w, block_rows=2000):
    rows = pre.shape[0]
    return pl.pallas_call(
        _rki_body,
        grid=(rows // block_rows,),
        in_specs=[
            pl.BlockSpec((block_rows, _D), lambda i: (i, 0)),
            pl.BlockSpec((block_rows, _D), lambda i: (i, 0)),
            pl.BlockSpec((_D, _D), lambda i: (0, 0)),
        ],
        out_specs=pl.BlockSpec((block_rows, _D), lambda i: (i, 0)),
        out_shape=jax.ShapeDtypeStruct((rows, _D), jnp.float32),
    )(pre, mk, w)


def _bond_upd_body(pre_z_ref, pre_m_ref, s_ref, r_ref, wz_ref, uw_ref, o_ref):
    s = s_ref[...]
    z = jax.nn.sigmoid(pre_z_ref[...] + jnp.dot(s, wz_ref[...],
                                                preferred_element_type=jnp.float32))
    m = jnp.tanh(pre_m_ref[...] + jnp.dot(r_ref[...], uw_ref[...],
                                          preferred_element_type=jnp.float32))
    o_ref[...] = (1.0 - z) * s + z * m


def _bond_upd(pre_z, pre_m, s, r, wz, uw, block_rows=2000):
    rows = pre_z.shape[0]
    bs = lambda: pl.BlockSpec((block_rows, _D), lambda i: (i, 0))
    return pl.pallas_call(
        _bond_upd_body,
        grid=(rows // block_rows,),
        in_specs=[bs(), bs(), bs(), bs(),
                  pl.BlockSpec((_D, _D), lambda i: (0, 0)),
                  pl.BlockSpec((_D, _D), lambda i: (0, 0))],
        out_specs=bs(),
        out_shape=jax.ShapeDtypeStruct((rows, _D), jnp.float32),
    )(pre_z, pre_m, s, r, wz, uw)


def _node_upd_body(pre_ref, mn_ref, ag_ref, u1_ref, u2_ref, o_ref):
    acc = pre_ref[...]
    acc = acc + jnp.dot(mn_ref[...], u1_ref[...], preferred_element_type=jnp.float32)
    acc = acc + jnp.dot(ag_ref[...], u2_ref[...], preferred_element_type=jnp.float32)
    o_ref[...] = _hswish(acc)


def _node_upd(pre_n, mn, aggr, u1, u2, block_rows=2000):
    rows = pre_n.shape[0]
    bs = lambda: pl.BlockSpec((block_rows, _D), lambda i: (i, 0))
    return pl.pallas_call(
        _node_upd_body,
        grid=(rows // block_rows,),
        in_specs=[bs(), bs(), bs(),
                  pl.BlockSpec((_D, _D), lambda i: (0, 0)),
                  pl.BlockSpec((_D, _D), lambda i: (0, 0))],
        out_specs=bs(),
        out_shape=jax.ShapeDtypeStruct((rows, _D), jnp.float32),
    )(pre_n, mn, aggr, u1, u2)


def _sc_gather(table, idx):
    K = idx.shape[0]
    D = table.shape[1]
    PW = K // _NW
    assert K % _NW == 0 and PW % 8 == 0, (K, PW)
    CH = min(128, PW)
    NFULL = PW // CH
    TAIL = PW - NFULL * CH
    assert TAIL % 8 == 0

    mesh = plsc.VectorSubcoreMesh(core_axis_name="c", subcore_axis_name="s")

    @functools.partial(
        pl.kernel, mesh=mesh,
        out_type=jax.ShapeDtypeStruct((K, D), jnp.float32),
        scratch_types=[
            pltpu.VMEM((PW,), jnp.int32),
            pltpu.VMEM((2, CH, D), jnp.float32),
            pltpu.SemaphoreType.DMA,
            pltpu.SemaphoreType.DMA,
        ],
    )
    def k(table_hbm, idx_hbm, out_hbm, idx_v, rows_v, sem0, sem1):
        wid = lax.axis_index("s") * _NC + lax.axis_index("c")
        base = wid * PW
        pltpu.sync_copy(idx_hbm.at[pl.ds(base, PW)], idx_v)
        sems = (sem0, sem1)

        def fire(c, b):
            pltpu.async_copy(table_hbm.at[idx_v.at[pl.ds(c * CH, CH)]],
                             rows_v.at[b], sems[b])

        def wait_write(c, b):
            pltpu.make_async_copy(
                table_hbm.at[idx_v.at[pl.ds(c * CH, CH)]],
                rows_v.at[b], sems[b]).wait()
            pltpu.sync_copy(rows_v.at[b],
                            out_hbm.at[pl.ds(base + c * CH, CH)])

        fire(0, 0)
        for c in range(1, NFULL):
            fire(c, c & 1)
            wait_write(c - 1, (c - 1) & 1)
        wait_write(NFULL - 1, (NFULL - 1) & 1)
        if TAIL:
            pltpu.async_copy(
                table_hbm.at[idx_v.at[pl.ds(NFULL * CH, TAIL)]],
                rows_v.at[1, pl.ds(0, TAIL)], sem1).wait()
            pltpu.sync_copy(rows_v.at[1, pl.ds(0, TAIL)],
                            out_hbm.at[pl.ds(base + NFULL * CH, TAIL)])

    return k(table, idx)


def _gather_rows(table, idx):
    return _sc_gather(table, idx)


def _segment_sum(vals, idx, num_segments):
    return jax.ops.segment_sum(vals, idx, num_segments=num_segments)


def kernel(node, connect, bond, bond_neighbour, W_node_w, W_node_b,
           W_node_final_w, W_node_final_b, W_bond_w, W_bond_b,
           W_bond_final_w, W_bond_final_b, W_z_w, W_z_b, W_r_w, W_r_b,
           U_w, W_w, W_b, W_n_w, W_n_b, U_n_w):
    i_idx = connect[0]
    j_idx = connect[1]
    ij_idx = bond_neighbour[0]
    ki_idx = bond_neighbour[1]
    N = node.shape[0]
    E = bond.shape[0]
    FN = node.shape[1]
    FB = bond.shape[1]

    nodei = _gather_rows(node, i_idx)

    mess_bond = _mm_fused([nodei, bond], [W_bond_w[:FN], W_bond_w[FN:]],
                          W_bond_b, "hswish")
    mess_node = _mm_fused([node], [W_node_w], W_node_b, "hswish",
                          block_rows=2000)
    pre_z = _mm_fused([nodei, bond], [W_z_w[:FN], W_z_w[FN:FN + FB]],
                      W_z_b, "none")
    pre_m = _mm_fused([nodei, bond], [W_w[:FN], W_w[FN:]], W_b, "none")
    pre_n = _mm_fused([node], [W_n_w], W_n_b, "none", block_rows=2000)

    pre_r_e = _mm_fused([nodei, bond], [W_r_w[:FN], W_r_w[FN:FN + FB]],
                        W_r_b, "none")
    pre_r = _gather_rows(pre_r_e, ij_idx)

    wz2 = W_z_w[FN + FB:]
    wr2 = W_r_w[FN + FB:]
    un1 = U_n_w[:_D]
    un2 = U_n_w[_D:]

    for _ in range(_LAYER):
        mess_ki = _gather_rows(mess_bond, ki_idx)
        s_ij = _segment_sum(mess_ki, ij_idx, E)
        rmk = _rki_fused(pre_r, mess_ki, wr2)
        r_ij = _segment_sum(rmk, ij_idx, E)
        mess_bond = _bond_upd(pre_z, pre_m, s_ij, r_ij, wz2, U_w)
        aggr_node = _segment_sum(mess_bond, j_idx, N)
        mess_node = _node_upd(pre_n, mess_node, aggr_node, un1, un2)

    out_bond = _mm_fused([nodei, bond, mess_bond],
                         [W_bond_final_w[:FN], W_bond_final_w[FN:FN + FB],
                          W_bond_final_w[FN + FB:]],
                         W_bond_final_b, "hswish")
    out_node = _mm_fused([node, mess_node],
                         [W_node_final_w[:FN], W_node_final_w[FN:]],
                         W_node_final_b, "hswish", block_rows=2000)
    return (out_node, out_bond)
